# Initial kernel scaffold; baseline (speedup 1.0000x reference)
import jax
import jax.numpy as jnp
from jax.experimental import pallas as pl


def kernel(x, edge_index, weight, bias):
    raise NotImplementedError("write your pallas kernel here")


# speedup vs baseline: 17.5041x; 17.5041x over previous
import functools

import jax
import jax.numpy as jnp
from jax import lax
from jax.experimental import pallas as pl
from jax.experimental.pallas import tpu as pltpu
from jax.experimental.pallas import tpu_sc as plsc

_NC = 2
_NS = 16
_NW = _NC * _NS
_L = 16


def _sc_mesh():
    return plsc.VectorSubcoreMesh(core_axis_name="c", subcore_axis_name="s")


def _deg_partials(row, n):
    e = row.shape[0]
    epw = e // _NW
    dk = 80
    nch = epw // dk
    zr = 80
    nzc = n // zr

    @functools.partial(
        pl.kernel,
        out_type=jax.ShapeDtypeStruct((_NC, n, _L), jnp.float32),
        mesh=_sc_mesh(),
        compiler_params=pltpu.CompilerParams(use_tc_tiling_on_sc=False),
        scratch_types=[
            pltpu.VMEM((2, dk), jnp.int32),
            pltpu.VMEM((dk, _L), jnp.float32),
            pltpu.VMEM((zr, _L), jnp.float32),
            pltpu.VMEM_SHARED((n, _L), jnp.float32),
        ],
    )
    def deg_kernel(row_hbm, out_hbm, idx_v, ones_v, zeros_v, acc_sh):
        cid = lax.axis_index("c")
        sid = lax.axis_index("s")
        wid = cid * _NS + sid

        @pl.loop(0, dk)
        def _(i):
            ones_v[i, :] = jnp.ones((_L,), jnp.float32)

        @pl.loop(0, zr)
        def _(i):
            zeros_v[i, :] = jnp.zeros((_L,), jnp.float32)

        @pl.loop(sid, nzc, step=_NS)
        def _(j):
            pltpu.sync_copy(zeros_v, acc_sh.at[pl.ds(j * zr, zr)])

        plsc.subcore_barrier()

        @pl.loop(0, nch)
        def _(c):
            base = wid * epw + c * dk
            pltpu.sync_copy(row_hbm.at[pl.ds(base, dk)], idx_v.at[0])
            pltpu.sync_copy(ones_v, acc_sh.at[idx_v.at[0]], add=True)

        plsc.subcore_barrier()

        @pl.loop(sid, nzc, step=_NS)
        def _(j):
            pltpu.sync_copy(acc_sh.at[pl.ds(j * zr, zr)],
                            out_hbm.at[cid, pl.ds(j * zr, zr)])

    return deg_kernel(row)


def _edge_partials(hs, col, row):
    n, d = hs.shape
    e = col.shape[0]
    epw = e // _NW
    dk = 80
    nch = epw // dk
    zr = 80
    nzc = n // zr

    @functools.partial(
        pl.kernel,
        out_type=jax.ShapeDtypeStruct((_NC, n, d), jnp.float32),
        mesh=_sc_mesh(),
        scratch_types=[
            pltpu.VMEM((2, dk), jnp.int32),
            pltpu.VMEM((2, dk), jnp.int32),
            pltpu.VMEM((2, dk, d), jnp.float32),
            pltpu.VMEM((zr, d), jnp.float32),
            pltpu.VMEM_SHARED((n, d), jnp.float32),
            pltpu.SemaphoreType.DMA,
        ],
    )
    def pump_kernel(hs_hbm, col_hbm, row_hbm, out_hbm,
                    colv, rowv, rows_v, zeros_v, acc_sh, sem):
        cid = lax.axis_index("c")
        sid = lax.axis_index("s")
        wid = cid * _NS + sid

        @pl.loop(0, zr)
        def _(i):
            @pl.loop(0, d // _L)
            def _(j):
                zeros_v[i, pl.ds(j * _L, _L)] = jnp.zeros((_L,), jnp.float32)

        @pl.loop(sid, nzc, step=_NS)
        def _(j):
            pltpu.sync_copy(zeros_v, acc_sh.at[pl.ds(j * zr, zr)])

        plsc.subcore_barrier()

        @pl.loop(0, nch)
        def _(c):
            base = wid * epw + c * dk
            pltpu.sync_copy(col_hbm.at[pl.ds(base, dk)], colv.at[0])
            pltpu.sync_copy(row_hbm.at[pl.ds(base, dk)], rowv.at[0])
            pltpu.async_copy(hs_hbm.at[colv.at[0]], rows_v.at[0], sem).wait()
            pltpu.sync_copy(rows_v.at[0], acc_sh.at[rowv.at[0]], add=True)

        plsc.subcore_barrier()

        @pl.loop(sid, nzc, step=_NS)
        def _(j):
            pltpu.sync_copy(acc_sh.at[pl.ds(j * zr, zr)],
                            out_hbm.at[cid, pl.ds(j * zr, zr)])

    return pump_kernel(hs, col, row)


def _matmul(x, weight):
    n, din = x.shape
    dout = weight.shape[1]
    blk = 1000

    def body(x_ref, w_ref, o_ref):
        o_ref[...] = jnp.dot(x_ref[...], w_ref[...],
                             preferred_element_type=jnp.float32)

    return pl.pallas_call(
        body,
        grid=(n // blk,),
        in_specs=[
            pl.BlockSpec((blk, din), lambda i: (i, 0)),
            pl.BlockSpec((din, dout), lambda i: (0, 0)),
        ],
        out_specs=pl.BlockSpec((blk, dout), lambda i: (i, 0)),
        out_shape=jax.ShapeDtypeStruct((n, dout), jnp.float32),
    )(x, weight)


def _scale(h, degp):
    n, d = h.shape
    blk = 1000

    def body(h_ref, d_ref, o_ref):
        deg = d_ref[0, :, 0:1] + d_ref[1, :, 0:1] + 1.0
        o_ref[...] = h_ref[...] * lax.rsqrt(deg)

    return pl.pallas_call(
        body,
        grid=(n // blk,),
        in_specs=[
            pl.BlockSpec((blk, d), lambda i: (i, 0)),
            pl.BlockSpec((_NC, blk, _L), lambda i: (0, i, 0)),
        ],
        out_specs=pl.BlockSpec((blk, d), lambda i: (i, 0)),
        out_shape=jax.ShapeDtypeStruct((n, d), jnp.float32),
    )(h, degp)


def _finish(hs, accp, degp, bias):
    n, d = hs.shape
    blk = 1000

    def body(hs_ref, a_ref, d_ref, b_ref, o_ref):
        deg = d_ref[0, :, 0:1] + d_ref[1, :, 0:1] + 1.0
        nrm = lax.rsqrt(deg)
        o_ref[...] = nrm * (a_ref[0] + a_ref[1] + hs_ref[...]) + b_ref[...]

    return pl.pallas_call(
        body,
        grid=(n // blk,),
        in_specs=[
            pl.BlockSpec((blk, d), lambda i: (i, 0)),
            pl.BlockSpec((_NC, blk, d), lambda i: (0, i, 0)),
            pl.BlockSpec((_NC, blk, _L), lambda i: (0, i, 0)),
            pl.BlockSpec((1, d), lambda i: (0, 0)),
        ],
        out_specs=pl.BlockSpec((blk, d), lambda i: (i, 0)),
        out_shape=jax.ShapeDtypeStruct((n, d), jnp.float32),
    )(hs, accp, degp, bias.reshape(1, d))


def kernel(x, edge_index, weight, bias):
    row = edge_index[0]
    col = edge_index[1]
    n = x.shape[0]
    degp = _deg_partials(row, n)
    h = _matmul(x, weight)
    hs = _scale(h, degp)
    accp = _edge_partials(hs, col, row)
    return _finish(hs, accp, degp, bias)
